# baseline (speedup 1.0000x reference)
import jax
import jax.numpy as jnp
from jax.experimental import pallas as pl
from jax.experimental.pallas import tpu as pltpu

_N = 10000
_D_IN = 128
_D_OUT = 128
_BM = 200
_NBLK = _N // _BM
_DEPTH = 4


def _a_copy(a_hbm, a_buf, sems, block, slot):
    return pltpu.make_async_copy(
        a_hbm.at[pl.ds(block * _BM, _BM), :], a_buf.at[slot], sems.at[slot]
    )


def _gcn_kernel(x_ref, w_ref, b_ref, a_hbm, out_ref, temple_ref, a_buf, sems):
    for k in range(_DEPTH):
        _a_copy(a_hbm, a_buf, sems, k, k).start()

    temple_ref[...] = jnp.dot(
        x_ref[...], w_ref[...],
        preferred_element_type=jnp.float32,
        precision=jax.lax.Precision.DEFAULT,
    )

    def body(i, carry):
        slot = jax.lax.rem(i, _DEPTH)
        _a_copy(a_hbm, a_buf, sems, i, slot).wait()
        out_ref[pl.ds(i * _BM, _BM), :] = (
            jnp.dot(
                a_buf[slot], temple_ref[...],
                preferred_element_type=jnp.float32,
                precision=jax.lax.Precision.DEFAULT,
            )
            + b_ref[...]
        )

        @pl.when(i + _DEPTH < _NBLK)
        def _refill():
            _a_copy(a_hbm, a_buf, sems, i + _DEPTH, slot).start()

        return carry

    jax.lax.fori_loop(0, _NBLK, body, 0)


def kernel(adjacency, features_matrix, weights, bias):
    bias2d = bias.reshape(1, _D_OUT)
    return pl.pallas_call(
        _gcn_kernel,
        in_specs=[
            pl.BlockSpec(memory_space=pltpu.MemorySpace.VMEM),
            pl.BlockSpec(memory_space=pltpu.MemorySpace.VMEM),
            pl.BlockSpec(memory_space=pltpu.MemorySpace.VMEM),
            pl.BlockSpec(memory_space=pltpu.MemorySpace.HBM),
        ],
        out_specs=pl.BlockSpec(memory_space=pltpu.MemorySpace.VMEM),
        out_shape=jax.ShapeDtypeStruct((_N, _D_OUT), jnp.float32),
        scratch_shapes=[
            pltpu.VMEM((_N, _D_IN), jnp.float32),
            pltpu.VMEM((_DEPTH, _BM, _N), jnp.float32),
            pltpu.SemaphoreType.DMA((_DEPTH,)),
        ],
    )(features_matrix, weights, bias2d, adjacency)


# speedup vs baseline: 1.0108x; 1.0108x over previous
import jax
import jax.numpy as jnp
from jax.experimental import pallas as pl
from jax.experimental.pallas import tpu as pltpu

_N = 10000
_D_IN = 128
_D_OUT = 128
_BM = 200
_NBLK = _N // _BM
_NPAIR = _NBLK // 2


def _a_copy(a_hbm, buf, sems, block, slot):
    return pltpu.make_async_copy(
        a_hbm.at[pl.ds(block * _BM, _BM), :], buf.at[slot], sems.at[slot]
    )


def _gcn_kernel(x_ref, w_ref, b_ref, a_hbm, out_ref,
                temple_ref, a0_buf, a1_buf, sems0, sems1):
    for k in range(2):
        _a_copy(a_hbm, a0_buf, sems0, 2 * k, k).start()
        _a_copy(a_hbm, a1_buf, sems1, 2 * k + 1, k).start()

    temple_ref[...] = jnp.dot(
        x_ref[...], w_ref[...],
        preferred_element_type=jnp.float32,
        precision=jax.lax.Precision.DEFAULT,
    )

    def _dot_bias(a_block):
        return (
            jnp.dot(
                a_block, temple_ref[...],
                preferred_element_type=jnp.float32,
                precision=jax.lax.Precision.DEFAULT,
            )
            + b_ref[...]
        )

    def body(j, carry):
        s = jax.lax.rem(j, 2)
        even, odd = 2 * j, 2 * j + 1

        _a_copy(a_hbm, a0_buf, sems0, even, s).wait()
        out_ref[pl.ds(even * _BM, _BM), :] = _dot_bias(a0_buf[s])

        @pl.when(even + 4 < _NBLK)
        def _refill_even():
            _a_copy(a_hbm, a0_buf, sems0, even + 4, s).start()

        _a_copy(a_hbm, a1_buf, sems1, odd, s).wait()
        out_ref[pl.ds(odd * _BM, _BM), :] = _dot_bias(a1_buf[s])

        @pl.when(odd + 4 < _NBLK)
        def _refill_odd():
            _a_copy(a_hbm, a1_buf, sems1, odd + 4, s).start()

        return carry

    jax.lax.fori_loop(0, _NPAIR, body, 0)


def kernel(adjacency, features_matrix, weights, bias):
    bias2d = bias.reshape(1, _D_OUT)
    return pl.pallas_call(
        _gcn_kernel,
        in_specs=[
            pl.BlockSpec(memory_space=pltpu.MemorySpace.VMEM),
            pl.BlockSpec(memory_space=pltpu.MemorySpace.VMEM),
            pl.BlockSpec(memory_space=pltpu.MemorySpace.VMEM),
            pl.BlockSpec(memory_space=pltpu.MemorySpace.HBM),
        ],
        out_specs=pl.BlockSpec(memory_space=pltpu.MemorySpace.VMEM),
        out_shape=jax.ShapeDtypeStruct((_N, _D_OUT), jnp.float32),
        scratch_shapes=[
            pltpu.VMEM((_N, _D_IN), jnp.float32),
            pltpu.VMEM((2, _BM, _N), jnp.float32),
            pltpu.VMEM((2, _BM, _N), jnp.float32),
            pltpu.SemaphoreType.DMA((2,)),
            pltpu.SemaphoreType.DMA((2,)),
        ],
    )(features_matrix, weights, bias2d, adjacency)


# speedup vs baseline: 1.0277x; 1.0168x over previous
import jax
import jax.numpy as jnp
from jax.experimental import pallas as pl
from jax.experimental.pallas import tpu as pltpu

_N = 10000
_D_IN = 128
_D_OUT = 128
_BM = 200
_NBLK = _N // _BM
_DEPTH = 3


def _a_copy(a_hbm, a_buf, sems, block, slot):
    return pltpu.make_async_copy(
        a_hbm.at[pl.ds(block * _BM, _BM), :], a_buf.at[slot], sems.at[slot]
    )


def _o_copy(o_stage, out_hbm, sems, block, slot):
    return pltpu.make_async_copy(
        o_stage.at[slot], out_hbm.at[pl.ds(block * _BM, _BM), :], sems.at[slot]
    )


def _gcn_kernel(x_ref, w_ref, b_ref, a_hbm, out_hbm,
                temple_ref, a_buf, o_stage, a_sems, o_sems):
    for k in range(_DEPTH):
        _a_copy(a_hbm, a_buf, a_sems, k, k).start()

    temple_ref[...] = jnp.dot(
        x_ref[...], w_ref[...],
        preferred_element_type=jnp.float32,
        precision=jax.lax.Precision.DEFAULT,
    )

    def body(i, carry):
        slot = jax.lax.rem(i, _DEPTH)
        oslot = jax.lax.rem(i, 2)

        @pl.when(i >= 2)
        def _drain_out():
            _o_copy(o_stage, out_hbm, o_sems, i - 2, oslot).wait()

        _a_copy(a_hbm, a_buf, a_sems, i, slot).wait()
        o_stage[oslot] = (
            jnp.dot(
                a_buf[slot], temple_ref[...],
                preferred_element_type=jnp.float32,
                precision=jax.lax.Precision.DEFAULT,
            )
            + b_ref[...]
        )

        @pl.when(i + _DEPTH < _NBLK)
        def _refill():
            _a_copy(a_hbm, a_buf, a_sems, i + _DEPTH, slot).start()

        _o_copy(o_stage, out_hbm, o_sems, i, oslot).start()
        return carry

    jax.lax.fori_loop(0, _NBLK, body, 0)

    for k in (_NBLK - 2, _NBLK - 1):
        _o_copy(o_stage, out_hbm, o_sems, k, k % 2).wait()


def kernel(adjacency, features_matrix, weights, bias):
    bias2d = bias.reshape(1, _D_OUT)
    return pl.pallas_call(
        _gcn_kernel,
        in_specs=[
            pl.BlockSpec(memory_space=pltpu.MemorySpace.VMEM),
            pl.BlockSpec(memory_space=pltpu.MemorySpace.VMEM),
            pl.BlockSpec(memory_space=pltpu.MemorySpace.VMEM),
            pl.BlockSpec(memory_space=pltpu.MemorySpace.HBM),
        ],
        out_specs=pl.BlockSpec(memory_space=pltpu.MemorySpace.HBM),
        out_shape=jax.ShapeDtypeStruct((_N, _D_OUT), jnp.float32),
        scratch_shapes=[
            pltpu.VMEM((_N, _D_IN), jnp.float32),
            pltpu.VMEM((_DEPTH, _BM, _N), jnp.float32),
            pltpu.VMEM((2, _BM, _D_OUT), jnp.float32),
            pltpu.SemaphoreType.DMA((_DEPTH,)),
            pltpu.SemaphoreType.DMA((2,)),
        ],
    )(features_matrix, weights, bias2d, adjacency)
